# baseline (speedup 1.0000x reference)
import jax
import jax.numpy as jnp
from jax import lax
from jax.experimental import pallas as pl
from jax.experimental.pallas import tpu as pltpu
from jax.experimental.pallas import tpu_sc as plsc

B = 4096
L = 50
LP = 56
EMB = 128
LANES = 16
NJ = EMB // LANES

NC = 2
NS = 16
NW = NC * NS
RPW = B // NW

NBUF = 8
TU = 2


def _fire(table_hbm, idx_v, r, nch, rows_b, sem_b):
    def issue2(c2, carry):
        for h in range(2):
            c = 2 * c2 + h
            pltpu.async_copy(
                table_hbm.at[idx_v.at[r, pl.ds(TU * c, TU)]],
                rows_b.at[pl.ds(TU * c, TU)], sem_b)
        return carry
    lax.fori_loop(0, nch // 2, issue2, 0)

    @pl.when(nch % 2 == 1)
    def _():
        c = nch - 1
        pltpu.async_copy(
            table_hbm.at[idx_v.at[r, pl.ds(TU * c, TU)]],
            rows_b.at[pl.ds(TU * c, TU)], sem_b)


def _body(x_hbm, lens_hbm, table_hbm, out_hbm,
          idx_v, lens_v, rows_bufs, out_v, sems):
    wid = lax.axis_index("s") * NC + lax.axis_index("c")
    base = wid * RPW

    pltpu.sync_copy(x_hbm.at[pl.ds(base, RPW)], idx_v)
    pltpu.sync_copy(lens_hbm.at[pl.ds(base, RPW)], lens_v.at[pl.ds(0, RPW)])

    def nchunks(len_r):
        return (len_r + (TU - 1)) // TU

    lens16 = lens_v[pl.ds(0, LANES)]
    for r in range(NBUF):
        _fire(table_hbm, idx_v, r, nchunks(lens16[r]),
              rows_bufs[r], sems[r])

    def group(gg, carry):
        lens_cur = lens_v[pl.ds(gg * LANES, LANES)]
        lens_pf = lens_v[pl.ds(gg * LANES + NBUF, LANES)]
        for rr in range(LANES):
            r = gg * LANES + rr
            b = rr % NBUF
            rows_b = rows_bufs[b]
            sem_b = sems[b]

            len_r = lens_cur[rr]
            nch = nchunks(len_r)

            def drain2(c2, carry, rows_b=rows_b, sem_b=sem_b):
                pltpu.make_async_copy(
                    table_hbm.at[pl.ds(0, 2 * TU)],
                    rows_b.at[pl.ds(0, 2 * TU)], sem_b).wait()
                return carry
            lax.fori_loop(0, nch // 2, drain2, 0)

            @pl.when(nch % 2 == 1)
            def _(rows_b=rows_b, sem_b=sem_b):
                pltpu.make_async_copy(
                    table_hbm.at[pl.ds(0, TU)],
                    rows_b.at[pl.ds(0, TU)], sem_b).wait()

            zeros = jnp.zeros((LANES,), jnp.float32)
            len_f = jnp.full((LANES,), len_r.astype(jnp.float32))
            inv = jnp.where(
                len_r > 0, jnp.full((LANES,), 1.0) / len_f, zeros)

            def acc_step(l, acc, rows_b=rows_b, len_r=len_r, inv=inv,
                         zeros=zeros):
                acc = list(acc)
                for k in range(TU):
                    t = TU * l + k
                    w = jnp.where(t < len_r, inv, zeros)
                    for j in range(NJ):
                        acc[j] = acc[j] + w * rows_b[
                            t, pl.ds(LANES * j, LANES)]
                return tuple(acc)

            acc = lax.fori_loop(
                0, nch, acc_step, tuple(zeros for _ in range(NJ)))

            for j in range(NJ):
                out_v[r, pl.ds(LANES * j, LANES)] = acc[j]

            @pl.when(r + NBUF < RPW)
            def _(rows_b=rows_b, sem_b=sem_b, r=r, rr=rr,
                  lens_pf=lens_pf):
                _fire(table_hbm, idx_v, r + NBUF,
                      nchunks(lens_pf[rr]), rows_b, sem_b)
        return carry

    lax.fori_loop(0, RPW // LANES, group, 0)

    pltpu.sync_copy(out_v, out_hbm.at[pl.ds(base, RPW)])


@jax.jit
def kernel(x, x_lens, table):
    xp = jnp.pad(x.astype(jnp.int32), ((0, 0), (0, LP - L)))
    lens = x_lens.astype(jnp.int32)

    mesh = plsc.VectorSubcoreMesh(
        core_axis_name="c", subcore_axis_name="s",
        num_cores=NC, num_subcores=NS)

    def body(x_hbm, lens_hbm, table_hbm, out_hbm,
             idx_v, lens_v, *rest):
        rows_bufs = rest[:NBUF]
        out_v = rest[NBUF]
        sems = rest[NBUF + 1:]
        _body(x_hbm, lens_hbm, table_hbm, out_hbm,
              idx_v, lens_v, rows_bufs, out_v, sems)

    f = pl.kernel(
        body,
        out_type=jax.ShapeDtypeStruct((B, EMB), jnp.float32),
        mesh=mesh,
        scratch_types=(
            [pltpu.VMEM((RPW, LP), jnp.int32),
             pltpu.VMEM((RPW + LANES, ), jnp.int32)]
            + [pltpu.VMEM((LP, EMB), jnp.float32)] * NBUF
            + [pltpu.VMEM((RPW, EMB), jnp.float32)]
            + [pltpu.SemaphoreType.DMA] * NBUF
        ),
    )
    return f(xp, lens, table)


# speedup vs baseline: 1.1880x; 1.1880x over previous
import jax
import jax.numpy as jnp
from jax import lax
from jax.experimental import pallas as pl
from jax.experimental.pallas import tpu as pltpu
from jax.experimental.pallas import tpu_sc as plsc

B = 4096
L = 50
LP = 56
EMB = 128
LANES = 16
NJ = EMB // LANES

NC = 2
NS = 16
NW = NC * NS
RPW = B // NW

NBUF = 8
TU = 2


def _fire(table_hbm, idx_v, r, nch, rows_b, sem_b):
    def issue(c, carry):
        pltpu.async_copy(
            table_hbm.at[idx_v.at[r, pl.ds(TU * c, TU)]],
            rows_b.at[pl.ds(TU * c, TU)], sem_b)
        return carry
    lax.fori_loop(0, nch, issue, 0)


def _body(x_hbm, lens_hbm, table_hbm, out_hbm,
          idx_v, lens_v, rows_bufs, out_v, sems):
    wid = lax.axis_index("s") * NC + lax.axis_index("c")
    base = wid * RPW

    pltpu.sync_copy(x_hbm.at[pl.ds(base, RPW)], idx_v)
    pltpu.sync_copy(lens_hbm.at[pl.ds(base, RPW)], lens_v.at[pl.ds(0, RPW)])
    lens_v[pl.ds(RPW, LANES)] = jnp.zeros((LANES,), jnp.int32)

    def nchunks(len_r):
        return (len_r + (TU - 1)) // TU

    lens16 = lens_v[pl.ds(0, LANES)]
    for r in range(NBUF):
        _fire(table_hbm, idx_v, r, nchunks(lens16[r]),
              rows_bufs[r], sems[r])

    def group(gg, carry):
        lens_cur = lens_v[pl.ds(gg * LANES, LANES)]
        lens_pf = lens_v[pl.ds(gg * LANES + NBUF, LANES)]
        for rr in range(LANES):
            r = gg * LANES + rr
            b = rr % NBUF
            rows_b = rows_bufs[b]
            sem_b = sems[b]

            len_r = lens_cur[rr]
            nch = nchunks(len_r)

            def drain(c, carry, rows_b=rows_b, sem_b=sem_b):
                pltpu.make_async_copy(
                    table_hbm.at[pl.ds(0, TU)],
                    rows_b.at[pl.ds(0, TU)], sem_b).wait()
                return carry
            lax.fori_loop(0, nch, drain, 0)

            zeros = jnp.zeros((LANES,), jnp.float32)
            len_f = jnp.full((LANES,), len_r.astype(jnp.float32))
            inv = jnp.where(
                len_r > 0, jnp.full((LANES,), 1.0) / len_f, zeros)

            def acc_step(l, acc, rows_b=rows_b, len_r=len_r, inv=inv,
                         zeros=zeros):
                acc = list(acc)
                for k in range(TU):
                    t = TU * l + k
                    w = jnp.where(t < len_r, inv, zeros)
                    for j in range(NJ):
                        acc[j] = acc[j] + w * rows_b[
                            t, pl.ds(LANES * j, LANES)]
                return tuple(acc)

            acc = lax.fori_loop(
                0, nch, acc_step, tuple(zeros for _ in range(NJ)))

            for j in range(NJ):
                out_v[r, pl.ds(LANES * j, LANES)] = acc[j]

            _fire(table_hbm, idx_v, r + NBUF,
                  nchunks(lens_pf[rr]), rows_b, sem_b)
        return carry

    lax.fori_loop(0, RPW // LANES, group, 0)

    pltpu.sync_copy(out_v, out_hbm.at[pl.ds(base, RPW)])


@jax.jit
def kernel(x, x_lens, table):
    xp = jnp.pad(x.astype(jnp.int32), ((0, 0), (0, LP - L)))
    lens = x_lens.astype(jnp.int32)

    mesh = plsc.VectorSubcoreMesh(
        core_axis_name="c", subcore_axis_name="s",
        num_cores=NC, num_subcores=NS)

    def body(x_hbm, lens_hbm, table_hbm, out_hbm,
             idx_v, lens_v, *rest):
        rows_bufs = rest[:NBUF]
        out_v = rest[NBUF]
        sems = rest[NBUF + 1:]
        _body(x_hbm, lens_hbm, table_hbm, out_hbm,
              idx_v, lens_v, rows_bufs, out_v, sems)

    f = pl.kernel(
        body,
        out_type=jax.ShapeDtypeStruct((B, EMB), jnp.float32),
        mesh=mesh,
        scratch_types=(
            [pltpu.VMEM((RPW, LP), jnp.int32),
             pltpu.VMEM((RPW + LANES, ), jnp.int32)]
            + [pltpu.VMEM((LP, EMB), jnp.float32)] * NBUF
            + [pltpu.VMEM((RPW, EMB), jnp.float32)]
            + [pltpu.SemaphoreType.DMA] * NBUF
        ),
    )
    return f(xp, lens, table)
